# baseline (speedup 1.0000x reference)
import functools

import jax
import jax.numpy as jnp
from jax import lax
from jax.experimental import pallas as pl
from jax.experimental.pallas import tpu as pltpu
from jax.experimental.pallas import tpu_sc as plsc

N = 10000
E = 320000
D_IN = 128
R1 = 8
R2 = 16
N1 = 32

NC = 2
NS = 16
NW = NC * NS

C = 64
CH0 = 64
CH1 = 256
STG = 32
NBUF = 4
E_PAD = NS * (CH0 + CH1) * C

N_P = 10016
N_ACC = 10240
ROWS_T = N_ACC // NS
PAD_ROW = 10008


def _tc_layer1_body(acc_ref, x_ref, wrel_ref, b_ref, wroot_ref, h1_ref):
    agg = acc_ref[0, :N_P, :] + acc_ref[1, :N_P, :]
    h1_ref[...] = jnp.maximum(
        jnp.dot(agg, wrel_ref[...], preferred_element_type=jnp.float32)
        + b_ref[...]
        + jnp.dot(x_ref[...], wroot_ref[...],
                  preferred_element_type=jnp.float32), 0.0)


def _tc_head_body(acc_ref, h1_ref, wrel_ref, b_ref, wroot_ref,
                  wfc1_ref, bfc1_ref, wfc2_ref, bfc2_ref, out_ref):
    agg = acc_ref[0, :N_P, :] + acc_ref[1, :N_P, :]
    h2 = jnp.maximum(
        jnp.dot(agg, wrel_ref[...], preferred_element_type=jnp.float32)
        + b_ref[...]
        + jnp.dot(h1_ref[...], wroot_ref[...],
                  preferred_element_type=jnp.float32), 0.0)
    h3 = jnp.maximum(
        jnp.dot(h2, wfc1_ref[...], preferred_element_type=jnp.float32)
        + bfc1_ref[...], 0.0)
    o = jnp.dot(h3, wfc2_ref[...], preferred_element_type=jnp.float32) \
        + bfc2_ref[...]
    valid = lax.broadcasted_iota(jnp.int32, (N_P, 1), 0) < N
    s = jnp.sum(jnp.where(valid, o, 0.0))
    out_ref[...] = o - s * (1.0 / N)


@functools.cache
def _make_sc_scatter_add(D):
    mesh = plsc.VectorSubcoreMesh(core_axis_name="c", subcore_axis_name="s",
                                  num_cores=NC, num_subcores=NS)

    @functools.partial(
        pl.kernel,
        out_type=jax.ShapeDtypeStruct((NC, N_ACC, D), jnp.float32),
        mesh=mesh,
        scratch_types=[
            pltpu.VMEM((STG, C), jnp.int32),
            pltpu.VMEM((STG, C), jnp.int32),
            [pltpu.VMEM((C, D), jnp.float32) for _ in range(NBUF)],
            [pltpu.SemaphoreType.DMA for _ in range(NBUF)],
            pltpu.VMEM_SHARED((N_ACC, D), jnp.float32),
        ],
        compiler_params=pltpu.CompilerParams(use_tc_tiling_on_sc=False),
    )
    def _sc_scatter_add(p_hbm, src_hbm, dst_hbm, zeros_hbm, out_hbm,
                        src_v, dst_v, rows, sems, acc_sh):
        c = lax.axis_index("c")
        s = lax.axis_index("s")
        row0 = s * ROWS_T

        pltpu.sync_copy(zeros_hbm.at[pl.ds(row0, ROWS_T)],
                        acc_sh.at[pl.ds(row0, ROWS_T)])
        plsc.subcore_barrier()

        base = jnp.where(c == 0, s * CH0, NS * CH0 + s * CH1)
        nstg = jnp.where(c == 0, CH0 // STG, CH1 // STG)

        def _stage(t, carry):
            sbase = base + t * STG
            pltpu.sync_copy(src_hbm.at[pl.ds(sbase, STG)], src_v)
            pltpu.sync_copy(dst_hbm.at[pl.ds(sbase, STG)], dst_v)
            for k in range(NBUF):
                pltpu.async_copy(p_hbm.at[src_v.at[k]], rows[k], sems[k])

            def _group(g, carry2):
                for k in range(NBUF):
                    q = g * NBUF + k
                    pltpu.make_async_copy(p_hbm.at[src_v.at[q]], rows[k],
                                          sems[k]).wait()
                    pltpu.sync_copy(rows[k], acc_sh.at[dst_v.at[q]], add=True)

                    @pl.when(q + NBUF < STG)
                    def _():
                        pltpu.async_copy(p_hbm.at[src_v.at[q + NBUF]],
                                         rows[k], sems[k])
                return carry2
            lax.fori_loop(0, STG // NBUF, _group, 0)
            return carry
        lax.fori_loop(0, nstg, _stage, 0)
        plsc.subcore_barrier()

        pltpu.sync_copy(acc_sh.at[pl.ds(row0, ROWS_T)],
                        out_hbm.at[c, pl.ds(row0, ROWS_T)])

    return _sc_scatter_add


def kernel(x, edge_index, W_rel1, b_rel1, W_root1, W_rel2, b_rel2, W_root2,
           W_fc1, b_fc1, W_fc2, b_fc2):
    ei = edge_index.astype(jnp.int32)
    ei = jnp.concatenate(
        [ei, jnp.full((2, E_PAD - E), PAD_ROW, jnp.int32)], axis=1)
    src2 = ei[0].reshape(NS * (CH0 + CH1), C)
    dst2 = ei[1].reshape(NS * (CH0 + CH1), C)

    x_pad = jnp.pad(x, ((0, N_P - N), (0, 0)))
    z128 = jnp.zeros((N_ACC, D_IN), jnp.float32)
    z8 = jnp.zeros((N_ACC, R1), jnp.float32)

    acc1 = _make_sc_scatter_add(D_IN)(x_pad, src2, dst2, z128)

    h1 = pl.pallas_call(
        _tc_layer1_body,
        out_shape=jax.ShapeDtypeStruct((N_P, R1), jnp.float32),
    )(acc1, x_pad, W_rel1, b_rel1.reshape(1, R1), W_root1)

    acc2 = _make_sc_scatter_add(R1)(h1, src2, dst2, z8)

    out = pl.pallas_call(
        _tc_head_body,
        out_shape=jax.ShapeDtypeStruct((N_P, 1), jnp.float32),
    )(acc2, h1, W_rel2, b_rel2.reshape(1, R2), W_root2,
      W_fc1, b_fc1.reshape(1, N1), W_fc2, b_fc2.reshape(1, 1))

    return out[:N]


# speedup vs baseline: 2.6034x; 2.6034x over previous
import functools

import jax
import jax.numpy as jnp
from jax import lax
from jax.experimental import pallas as pl
from jax.experimental.pallas import tpu as pltpu
from jax.experimental.pallas import tpu_sc as plsc

N = 10000
E = 320000
D_IN = 128
R1 = 8
R2 = 16
N1 = 32

NC = 2
NS = 16
NW = NC * NS

C = 64
CH = 160
STG = 32
NBUF = 4
E_PAD = NW * CH * C

N_P = 10016
N_B = N_P // NS
N_ACC = 10240
ROWS_T = N_ACC // NS
PAD_ROW = 10008


def _tc_layer1_body(acc_ref, x_ref, wrel_ref, b_ref, wroot_ref, h1_ref):
    agg = acc_ref[0, :N_P, :] + acc_ref[1, :N_P, :]
    h1_ref[...] = jnp.maximum(
        jnp.dot(agg, wrel_ref[...], preferred_element_type=jnp.float32)
        + b_ref[...]
        + jnp.dot(x_ref[...], wroot_ref[...],
                  preferred_element_type=jnp.float32), 0.0)


def _tc_head_body(acc_ref, h1_ref, wrel_ref, b_ref, wroot_ref,
                  wfc1_ref, bfc1_ref, wfc2_ref, bfc2_ref, out_ref):
    agg = acc_ref[0, :N_P, :] + acc_ref[1, :N_P, :]
    h2 = jnp.maximum(
        jnp.dot(agg, wrel_ref[...], preferred_element_type=jnp.float32)
        + b_ref[...]
        + jnp.dot(h1_ref[...], wroot_ref[...],
                  preferred_element_type=jnp.float32), 0.0)
    h3 = jnp.maximum(
        jnp.dot(h2, wfc1_ref[...], preferred_element_type=jnp.float32)
        + bfc1_ref[...], 0.0)
    o = jnp.dot(h3, wfc2_ref[...], preferred_element_type=jnp.float32) \
        + bfc2_ref[...]
    valid = lax.broadcasted_iota(jnp.int32, (N_P, 1), 0) < N
    s = jnp.sum(jnp.where(valid, o, 0.0))
    out_ref[...] = o - s * (1.0 / N)


@functools.cache
def _make_sc_scatter_add(D):
    DP = min(D, 64)
    NPASS = D // DP
    mesh = plsc.VectorSubcoreMesh(core_axis_name="c", subcore_axis_name="s",
                                  num_cores=NC, num_subcores=NS)

    @functools.partial(
        pl.kernel,
        out_type=jax.ShapeDtypeStruct((NC, N_ACC, D), jnp.float32),
        mesh=mesh,
        scratch_types=[
            pltpu.VMEM((STG, C), jnp.int32),
            pltpu.VMEM((STG, C), jnp.int32),
            [pltpu.VMEM((C, DP), jnp.float32) for _ in range(NBUF)],
            [pltpu.SemaphoreType.DMA for _ in range(NBUF)],
            pltpu.VMEM_SHARED((N_P, DP), jnp.float32),
            pltpu.VMEM_SHARED((N_ACC, DP), jnp.float32),
        ],
        compiler_params=pltpu.CompilerParams(use_tc_tiling_on_sc=False),
    )
    def _sc_scatter_add(tab_hbm, src_hbm, dst_hbm, zeros_hbm, out_hbm,
                        src_v, dst_v, rows, sems, tab_sh, acc_sh):
        c = lax.axis_index("c")
        s = lax.axis_index("s")
        row0 = s * ROWS_T
        wid = s * NC + c

        for f in range(NPASS):
            pltpu.sync_copy(tab_hbm.at[f, pl.ds(s * N_B, N_B)],
                            tab_sh.at[pl.ds(s * N_B, N_B)])
            pltpu.sync_copy(zeros_hbm.at[pl.ds(row0, ROWS_T)],
                            acc_sh.at[pl.ds(row0, ROWS_T)])
            plsc.subcore_barrier()

            def _stage(t, carry):
                sbase = wid * CH + t * STG
                pltpu.sync_copy(src_hbm.at[pl.ds(sbase, STG)], src_v)
                pltpu.sync_copy(dst_hbm.at[pl.ds(sbase, STG)], dst_v)
                for k in range(NBUF):
                    pltpu.async_copy(tab_sh.at[src_v.at[k]], rows[k], sems[k])

                def _group(g, carry2):
                    for k in range(NBUF):
                        q = g * NBUF + k
                        pltpu.make_async_copy(tab_sh.at[src_v.at[q]], rows[k],
                                              sems[k]).wait()
                        pltpu.sync_copy(rows[k], acc_sh.at[dst_v.at[q]],
                                        add=True)

                        @pl.when(q + NBUF < STG)
                        def _():
                            pltpu.async_copy(tab_sh.at[src_v.at[q + NBUF]],
                                             rows[k], sems[k])
                    return carry2
                lax.fori_loop(0, STG // NBUF, _group, 0)
                return carry
            lax.fori_loop(0, CH // STG, _stage, 0)
            plsc.subcore_barrier()

            pltpu.sync_copy(acc_sh.at[pl.ds(row0, ROWS_T)],
                            out_hbm.at[c, pl.ds(row0, ROWS_T),
                                       pl.ds(f * DP, DP)])

    return _sc_scatter_add


def kernel(x, edge_index, W_rel1, b_rel1, W_root1, W_rel2, b_rel2, W_root2,
           W_fc1, b_fc1, W_fc2, b_fc2):
    ei = edge_index.astype(jnp.int32)
    ei = jnp.concatenate(
        [ei, jnp.full((2, E_PAD - E), PAD_ROW, jnp.int32)], axis=1)
    src2 = ei[0].reshape(NW * CH, C)
    dst2 = ei[1].reshape(NW * CH, C)

    x_pad = jnp.pad(x, ((0, N_P - N), (0, 0)))
    x_tab = jnp.stack([x_pad[:, :64], x_pad[:, 64:]])
    z64 = jnp.zeros((N_ACC, 64), jnp.float32)
    z8 = jnp.zeros((N_ACC, R1), jnp.float32)

    acc1 = _make_sc_scatter_add(D_IN)(x_tab, src2, dst2, z64)

    h1 = pl.pallas_call(
        _tc_layer1_body,
        out_shape=jax.ShapeDtypeStruct((N_P, R1), jnp.float32),
    )(acc1, x_pad, W_rel1, b_rel1.reshape(1, R1), W_root1)

    acc2 = _make_sc_scatter_add(R1)(h1.reshape(1, N_P, R1), src2, dst2, z8)

    out = pl.pallas_call(
        _tc_head_body,
        out_shape=jax.ShapeDtypeStruct((N_P, 1), jnp.float32),
    )(acc2, h1, W_rel2, b_rel2.reshape(1, R2), W_root2,
      W_fc1, b_fc1.reshape(1, N1), W_fc2, b_fc2.reshape(1, 1))

    return out[:N]
